# baseline (speedup 1.0000x reference)
import functools

import jax
import jax.numpy as jnp
from jax import lax
from jax.experimental import pallas as pl
from jax.experimental.pallas import tpu as pltpu
from jax.experimental.pallas import tpu_sc as plsc


def _sc_gather(table, idx_flat, n_tokens, emb):
    info = plsc.get_sparse_core_info()
    nw = info.num_cores * info.num_subcores
    assert n_tokens % (8 * nw) == 0
    b_per_w = n_tokens // nw
    nc = info.num_cores

    mesh = plsc.VectorSubcoreMesh(core_axis_name="c", subcore_axis_name="s")

    @functools.partial(
        pl.kernel,
        mesh=mesh,
        out_type=jax.ShapeDtypeStruct((n_tokens, emb), jnp.float32),
        scratch_types=[
            pltpu.VMEM((b_per_w,), jnp.int32),
            pltpu.VMEM((b_per_w, emb), jnp.float32),
            pltpu.SemaphoreType.DMA,
        ],
    )
    def gather_kernel(table_hbm, idx_hbm, out_hbm, idx_v, rows_v, sem):
        wid = lax.axis_index("s") * nc + lax.axis_index("c")
        base = wid * b_per_w
        pltpu.sync_copy(idx_hbm.at[pl.ds(base, b_per_w)], idx_v)
        pltpu.async_copy(table_hbm.at[idx_v], rows_v, sem).wait()
        pltpu.sync_copy(rows_v, out_hbm.at[pl.ds(base, b_per_w)])

    return gather_kernel(table, idx_flat)


def _tc_matmul(x_tok, pos_emb, w_out, batch, seq, vt):
    n = batch * seq
    emb = x_tok.shape[1]
    vocab = w_out.shape[0]
    n_vt = pl.cdiv(vocab, vt)

    def mm_kernel(x_ref, pos_ref, w_ref, out_ref, xs_ref):
        @pl.when(pl.program_id(0) == 0)
        def _():
            for b in range(batch):
                xs_ref[b * seq:(b + 1) * seq, :] = (
                    x_ref[b * seq:(b + 1) * seq, :] + pos_ref[...]
                ).astype(jnp.bfloat16)

        b = pl.program_id(1)
        w_bf = w_ref[...].astype(jnp.bfloat16)
        out_ref[...] = lax.dot_general(
            xs_ref[pl.ds(b * seq, seq), :], w_bf,
            dimension_numbers=(((1,), (1,)), ((), ())),
            preferred_element_type=jnp.float32,
        )

    return pl.pallas_call(
        mm_kernel,
        grid=(n_vt, batch),
        in_specs=[
            pl.BlockSpec((n, emb), lambda v, b: (0, 0)),
            pl.BlockSpec((seq, emb), lambda v, b: (0, 0)),
            pl.BlockSpec((vt, emb), lambda v, b: (v, 0)),
        ],
        out_specs=pl.BlockSpec((None, seq, vt), lambda v, b: (b, 0, v)),
        out_shape=jax.ShapeDtypeStruct((batch, seq, vocab), jnp.float32),
        scratch_shapes=[pltpu.VMEM((n, emb), jnp.bfloat16)],
        compiler_params=pltpu.CompilerParams(
            dimension_semantics=("arbitrary", "arbitrary"),
        ),
    )(x_tok, pos_emb, w_out)


def kernel(in_idx, tok_emb, pos_emb, W_out):
    batch, seq = in_idx.shape
    vocab, emb = W_out.shape
    idx_flat = in_idx.reshape(-1)
    x_tok = _sc_gather(tok_emb, idx_flat, batch * seq, emb)
    return _tc_matmul(x_tok, pos_emb[:seq], W_out, batch, seq, vt=512)


# speedup vs baseline: 1.5622x; 1.5622x over previous
import functools

import jax
import jax.numpy as jnp
from jax import lax
from jax.experimental import pallas as pl
from jax.experimental.pallas import tpu as pltpu
from jax.experimental.pallas import tpu_sc as plsc


def _sc_gather(table, idx_flat, n_tokens, emb):
    info = plsc.get_sparse_core_info()
    nw = info.num_cores * info.num_subcores
    assert n_tokens % (8 * nw) == 0
    b_per_w = n_tokens // nw
    nc = info.num_cores

    mesh = plsc.VectorSubcoreMesh(core_axis_name="c", subcore_axis_name="s")

    @functools.partial(
        pl.kernel,
        mesh=mesh,
        out_type=jax.ShapeDtypeStruct((n_tokens, emb), jnp.float32),
        scratch_types=[
            pltpu.VMEM((b_per_w,), jnp.int32),
            pltpu.VMEM((b_per_w, emb), jnp.float32),
            pltpu.SemaphoreType.DMA,
        ],
    )
    def gather_kernel(table_hbm, idx_hbm, out_hbm, idx_v, rows_v, sem):
        wid = lax.axis_index("s") * nc + lax.axis_index("c")
        base = wid * b_per_w
        pltpu.sync_copy(idx_hbm.at[pl.ds(base, b_per_w)], idx_v)
        pltpu.async_copy(table_hbm.at[idx_v], rows_v, sem).wait()
        pltpu.sync_copy(rows_v, out_hbm.at[pl.ds(base, b_per_w)])

    return gather_kernel(table, idx_flat)


def _tc_matmul(x_tok, pos_emb, w_out, batch, seq, vt):
    n = batch * seq
    emb = x_tok.shape[1]
    vocab = w_out.shape[0]
    n_vt = pl.cdiv(vocab, vt)

    n_sh = seq // 128

    def mm_kernel(x_ref, pos_ref, w_ref, out_ref, xs_ref, wb_ref):
        v = pl.program_id(0)
        sh = pl.program_id(1)

        @pl.when((v == 0) & (sh == 0))
        def _():
            for bb in range(batch):
                xs_ref[bb * seq:(bb + 1) * seq, :] = (
                    x_ref[bb * seq:(bb + 1) * seq, :] + pos_ref[...]
                ).astype(jnp.bfloat16)

        @pl.when(sh == 0)
        def _():
            wb_ref[...] = w_ref[...].astype(jnp.bfloat16)

        for b in range(batch):
            out_ref[:, 0, b, :] = lax.dot_general(
                wb_ref[...], xs_ref[pl.ds(b * seq + sh * 128, 128), :],
                dimension_numbers=(((1,), (1,)), ((), ())),
                preferred_element_type=jnp.float32,
            )

    out4 = pl.pallas_call(
        mm_kernel,
        grid=(n_vt, n_sh),
        in_specs=[
            pl.BlockSpec((n, emb), lambda v, s: (0, 0)),
            pl.BlockSpec((seq, emb), lambda v, s: (0, 0)),
            pl.BlockSpec((vt, emb), lambda v, s: (v, 0)),
        ],
        out_specs=pl.BlockSpec(
            (vt, 1, batch, 128), lambda v, s: (v, s, 0, 0)),
        out_shape=jax.ShapeDtypeStruct((vocab, n_sh, batch, 128),
                                       jnp.float32),
        scratch_shapes=[
            pltpu.VMEM((n, emb), jnp.bfloat16),
            pltpu.VMEM((vt, emb), jnp.bfloat16),
        ],
        compiler_params=pltpu.CompilerParams(
            dimension_semantics=("arbitrary", "arbitrary"),
        ),
    )(x_tok, pos_emb, w_out)
    return out4.transpose(2, 1, 3, 0).reshape(batch, seq, vocab)


def kernel(in_idx, tok_emb, pos_emb, W_out):
    batch, seq = in_idx.shape
    vocab, emb = W_out.shape
    idx_flat = in_idx.reshape(-1)
    x_tok = _sc_gather(tok_emb, idx_flat, batch * seq, emb)
    return _tc_matmul(x_tok, pos_emb[:seq], W_out, batch, seq, vt=2048)


# speedup vs baseline: 3.2188x; 2.0604x over previous
import functools

import jax
import jax.numpy as jnp
from jax import lax
from jax.experimental import pallas as pl
from jax.experimental.pallas import tpu as pltpu
from jax.experimental.pallas import tpu_sc as plsc


def _sc_gather(table, idx_flat, n_tokens, emb):
    info = plsc.get_sparse_core_info()
    nw = info.num_cores * info.num_subcores
    assert n_tokens % (8 * nw) == 0
    b_per_w = n_tokens // nw
    nc = info.num_cores

    mesh = plsc.VectorSubcoreMesh(core_axis_name="c", subcore_axis_name="s")

    @functools.partial(
        pl.kernel,
        mesh=mesh,
        out_type=jax.ShapeDtypeStruct((n_tokens, emb), jnp.float32),
        scratch_types=[
            pltpu.VMEM((b_per_w,), jnp.int32),
            pltpu.VMEM((b_per_w, emb), jnp.float32),
            pltpu.SemaphoreType.DMA,
        ],
    )
    def gather_kernel(table_hbm, idx_hbm, out_hbm, idx_v, rows_v, sem):
        wid = lax.axis_index("s") * nc + lax.axis_index("c")
        base = wid * b_per_w
        pltpu.sync_copy(idx_hbm.at[pl.ds(base, b_per_w)], idx_v)
        pltpu.async_copy(table_hbm.at[idx_v], rows_v, sem).wait()
        pltpu.sync_copy(rows_v, out_hbm.at[pl.ds(base, b_per_w)])

    return gather_kernel(table, idx_flat)


def _tc_prep(x_tok, pos_emb, batch, seq):
    n = batch * seq
    emb = x_tok.shape[1]

    def prep_kernel(x_ref, pos_ref, xs_ref):
        for b in range(batch):
            xs_ref[b * seq:(b + 1) * seq, :] = (
                x_ref[b * seq:(b + 1) * seq, :] + pos_ref[...]
            ).astype(jnp.bfloat16)

    return pl.pallas_call(
        prep_kernel,
        out_shape=jax.ShapeDtypeStruct((n, emb), jnp.bfloat16),
    )(x_tok, pos_emb)


def _tc_matmul(xs, w_out, batch, seq, vt):
    n = batch * seq
    emb = xs.shape[1]
    vocab = w_out.shape[0]
    n_vt = pl.cdiv(vocab, vt)

    n_sh = seq // 128
    rem = vocab - (n_vt - 1) * vt

    def mm_kernel(xs_ref, w_ref, out_ref, wb_ref,
                  res_a, res_b, tail_a, tail_b, sem_a, sem_b):
        v = pl.program_id(0)
        b = pl.program_id(1)

        @pl.when(b == 0)
        def _():
            wb_ref[...] = w_ref[...].astype(jnp.bfloat16)

        res = lax.dot_general(
            wb_ref[...], xs_ref[pl.ds(b * seq, seq), :],
            dimension_numbers=(((1,), (1,)), ((), ())),
            preferred_element_type=jnp.float32,
        )

        def copies(buf, sem, rows):
            return [
                pltpu.make_async_copy(
                    buf.at[:, pl.ds(sh * 128, 128)],
                    out_ref.at[pl.ds(v * vt, rows), sh * 2 + b, :],
                    sem,
                )
                for sh in range(n_sh)
            ]

        def step(buf, sem, tail):
            @pl.when(v >= 1)
            def _():
                for c in copies(buf, sem, vt):
                    c.wait()

            @pl.when(v < n_vt - 1)
            def _():
                buf[...] = res
                for c in copies(buf, sem, vt):
                    c.start()

            @pl.when(v == n_vt - 1)
            def _():
                tail[...] = res[:rem, :]
                for c in copies(tail, sem, rem):
                    c.start()

        @pl.when(b == 0)
        def _():
            step(res_a, sem_a, tail_a)

        @pl.when(b == 1)
        def _():
            step(res_b, sem_b, tail_b)

        @pl.when((v == n_vt - 1) & (b == 1))
        def _():
            for c in copies(tail_a, sem_a, rem):
                c.wait()
            for c in copies(tail_b, sem_b, rem):
                c.wait()

    out3 = pl.pallas_call(
        mm_kernel,
        grid=(n_vt, batch),
        in_specs=[
            pl.BlockSpec((n, emb), lambda v, b: (0, 0)),
            pl.BlockSpec((vt, emb), lambda v, b: (v, 0)),
        ],
        out_specs=pl.BlockSpec(memory_space=pltpu.HBM),
        out_shape=jax.ShapeDtypeStruct((vocab, 2 * n_sh, 128),
                                       jnp.float32),
        scratch_shapes=[
            pltpu.VMEM((vt, emb), jnp.bfloat16),
            pltpu.VMEM((vt, seq), jnp.float32),
            pltpu.VMEM((vt, seq), jnp.float32),
            pltpu.VMEM((rem, seq), jnp.float32),
            pltpu.VMEM((rem, seq), jnp.float32),
            pltpu.SemaphoreType.DMA,
            pltpu.SemaphoreType.DMA,
        ],
        compiler_params=pltpu.CompilerParams(
            dimension_semantics=("arbitrary", "arbitrary"),
        ),
    )(xs, w_out)
    out4 = out3.reshape(vocab, n_sh, batch, 128)
    return out4.transpose(2, 1, 3, 0).reshape(batch, seq, vocab)


def kernel(in_idx, tok_emb, pos_emb, W_out):
    batch, seq = in_idx.shape
    vocab, emb = W_out.shape
    idx_flat = in_idx.reshape(-1)
    x_tok = _sc_gather(tok_emb, idx_flat, batch * seq, emb)
    xs = _tc_prep(x_tok, pos_emb[:seq], batch, seq)
    return _tc_matmul(xs, W_out, batch, seq, vt=512)


# speedup vs baseline: 3.6949x; 1.1479x over previous
import functools

import jax
import jax.numpy as jnp
from jax import lax
from jax.experimental import pallas as pl
from jax.experimental.pallas import tpu as pltpu
from jax.experimental.pallas import tpu_sc as plsc


def _sc_gather(table, idx_flat, n_tokens, emb):
    info = plsc.get_sparse_core_info()
    nw = info.num_cores * info.num_subcores
    assert n_tokens % (8 * nw) == 0
    b_per_w = n_tokens // nw
    nc = info.num_cores

    mesh = plsc.VectorSubcoreMesh(core_axis_name="c", subcore_axis_name="s")

    @functools.partial(
        pl.kernel,
        mesh=mesh,
        out_type=jax.ShapeDtypeStruct((n_tokens, emb), jnp.float32),
        scratch_types=[
            pltpu.VMEM((b_per_w,), jnp.int32),
            pltpu.VMEM((b_per_w, emb), jnp.float32),
            pltpu.SemaphoreType.DMA,
        ],
    )
    def gather_kernel(table_hbm, idx_hbm, out_hbm, idx_v, rows_v, sem):
        wid = lax.axis_index("s") * nc + lax.axis_index("c")
        base = wid * b_per_w
        pltpu.sync_copy(idx_hbm.at[pl.ds(base, b_per_w)], idx_v)
        pltpu.async_copy(table_hbm.at[idx_v], rows_v, sem).wait()
        pltpu.sync_copy(rows_v, out_hbm.at[pl.ds(base, b_per_w)])

    return gather_kernel(table, idx_flat)


def _tc_prep(x_tok, pos_emb, batch, seq):
    n = batch * seq
    emb = x_tok.shape[1]

    def prep_kernel(x_ref, pos_ref, xs_ref):
        for b in range(batch):
            xs_ref[b * seq:(b + 1) * seq, :] = (
                x_ref[b * seq:(b + 1) * seq, :] + pos_ref[...]
            ).astype(jnp.bfloat16)

    return pl.pallas_call(
        prep_kernel,
        out_shape=jax.ShapeDtypeStruct((n, emb), jnp.bfloat16),
    )(x_tok, pos_emb)


def _tc_matmul(xs, w_out, batch, seq, vt):
    n = batch * seq
    emb = xs.shape[1]
    vocab = w_out.shape[0]
    n_vt = pl.cdiv(vocab, vt)

    n_sh = seq // 128
    rem = vocab - (n_vt - 1) * vt

    def mm_kernel(xs_ref, w_ref, out_ref, wb_ref,
                  res_a, res_b, tail_a, tail_b, sem_a, sem_b):
        v = pl.program_id(0)
        b = pl.program_id(1)

        @pl.when(b == 0)
        def _():
            wb_ref[...] = w_ref[...].astype(jnp.bfloat16)

        res = lax.dot_general(
            wb_ref[...], xs_ref[pl.ds(b * seq, seq), :],
            dimension_numbers=(((1,), (1,)), ((), ())),
            preferred_element_type=jnp.float32,
        )

        def copies(buf, sem, rows):
            return [
                pltpu.make_async_copy(
                    buf.at[:, pl.ds(sh * 128, 128)],
                    out_ref.at[pl.ds(v * vt, rows), sh * 2 + b, :],
                    sem,
                )
                for sh in range(n_sh)
            ]

        def step(buf, sem, tail):
            @pl.when(v >= 1)
            def _():
                for c in copies(buf, sem, vt):
                    c.wait()

            @pl.when(v < n_vt - 1)
            def _():
                buf[...] = res
                for c in copies(buf, sem, vt):
                    c.start()

            @pl.when(v == n_vt - 1)
            def _():
                tail[...] = res[:rem, :]
                for c in copies(tail, sem, rem):
                    c.start()

        @pl.when(b == 0)
        def _():
            step(res_a, sem_a, tail_a)

        @pl.when(b == 1)
        def _():
            step(res_b, sem_b, tail_b)

        @pl.when((v == n_vt - 1) & (b == 1))
        def _():
            for c in copies(tail_a, sem_a, rem):
                c.wait()
            for c in copies(tail_b, sem_b, rem):
                c.wait()

    out3 = pl.pallas_call(
        mm_kernel,
        grid=(n_vt, batch),
        in_specs=[
            pl.BlockSpec((n, emb), lambda v, b: (0, 0)),
            pl.BlockSpec((vt, emb), lambda v, b: (v, 0)),
        ],
        out_specs=pl.BlockSpec(memory_space=pltpu.HBM),
        out_shape=jax.ShapeDtypeStruct((vocab, 2 * n_sh, 128),
                                       jnp.float32),
        scratch_shapes=[
            pltpu.VMEM((vt, emb), jnp.bfloat16),
            pltpu.VMEM((vt, seq), jnp.float32),
            pltpu.VMEM((vt, seq), jnp.float32),
            pltpu.VMEM((rem, seq), jnp.float32),
            pltpu.VMEM((rem, seq), jnp.float32),
            pltpu.SemaphoreType.DMA,
            pltpu.SemaphoreType.DMA,
        ],
        compiler_params=pltpu.CompilerParams(
            dimension_semantics=("arbitrary", "arbitrary"),
        ),
    )(xs, w_out)
    out4 = out3.reshape(vocab, n_sh, batch, 128)
    return out4.transpose(2, 1, 3, 0).reshape(batch, seq, vocab)


def kernel(in_idx, tok_emb, pos_emb, W_out):
    batch, seq = in_idx.shape
    vocab, emb = W_out.shape
    idx_flat = in_idx.reshape(-1)
    x_tok = _sc_gather(tok_emb, idx_flat, batch * seq, emb)
    xs = _tc_prep(x_tok, pos_emb[:seq], batch, seq)
    return _tc_matmul(xs, W_out, batch, seq, vt=1024)


# speedup vs baseline: 4.0055x; 1.0841x over previous
import functools

import jax
import jax.numpy as jnp
from jax import lax
from jax.experimental import pallas as pl
from jax.experimental.pallas import tpu as pltpu
from jax.experimental.pallas import tpu_sc as plsc


def _sc_gather(table, idx_flat, n_tokens, emb):
    info = plsc.get_sparse_core_info()
    nw = info.num_cores * info.num_subcores
    assert n_tokens % (8 * nw) == 0
    b_per_w = n_tokens // nw
    nc = info.num_cores

    mesh = plsc.VectorSubcoreMesh(core_axis_name="c", subcore_axis_name="s")

    @functools.partial(
        pl.kernel,
        mesh=mesh,
        out_type=jax.ShapeDtypeStruct((n_tokens, emb), jnp.float32),
        scratch_types=[
            pltpu.VMEM((b_per_w,), jnp.int32),
            pltpu.VMEM((b_per_w, emb), jnp.float32),
            pltpu.SemaphoreType.DMA,
        ],
    )
    def gather_kernel(table_hbm, idx_hbm, out_hbm, idx_v, rows_v, sem):
        wid = lax.axis_index("s") * nc + lax.axis_index("c")
        base = wid * b_per_w
        pltpu.sync_copy(idx_hbm.at[pl.ds(base, b_per_w)], idx_v)
        pltpu.async_copy(table_hbm.at[idx_v], rows_v, sem).wait()
        pltpu.sync_copy(rows_v, out_hbm.at[pl.ds(base, b_per_w)])

    return gather_kernel(table, idx_flat)


def _tc_prep(x_tok, pos_emb, batch, seq):
    n = batch * seq
    emb = x_tok.shape[1]

    def prep_kernel(x_ref, pos_ref, xs_ref):
        for b in range(batch):
            xs_ref[b * seq:(b + 1) * seq, :] = (
                x_ref[b * seq:(b + 1) * seq, :] + pos_ref[...]
            ).astype(jnp.bfloat16)

    return pl.pallas_call(
        prep_kernel,
        out_shape=jax.ShapeDtypeStruct((n, emb), jnp.bfloat16),
    )(x_tok, pos_emb)


def _tc_matmul(xs, w_out, batch, seq, vt):
    n = batch * seq
    emb = xs.shape[1]
    vocab = w_out.shape[0]
    n_vt = pl.cdiv(vocab, vt)

    n_sh = seq // 128
    rem = vocab - (n_vt - 1) * vt

    half = seq // 2

    def mm_kernel(xs_ref, w_ref, out_ref, wb2_ref,
                  res_a, res_b, tail_a, tail_b, sem_a, sem_b):
        v = pl.program_id(0)
        b = pl.program_id(1)

        @pl.when((v == 0) & (b == 0))
        def _():
            wb2_ref[pl.ds(0, vt), :] = w_ref[...].astype(jnp.bfloat16)

        @pl.when((b == 1) & (v < n_vt - 1))
        def _():
            wb2_ref[pl.ds(((v + 1) % 2) * vt, vt), :] = (
                w_ref[...].astype(jnp.bfloat16))

        def dot_half(h):
            return lax.dot_general(
                wb2_ref[pl.ds((v % 2) * vt, vt), :],
                xs_ref[pl.ds(b * seq + h * half, half), :],
                dimension_numbers=(((1,), (1,)), ((), ())),
                preferred_element_type=jnp.float32,
            )

        def copies(buf, sem, rows):
            return [
                pltpu.make_async_copy(
                    buf.at[:, pl.ds(sh * 128, 128)],
                    out_ref.at[pl.ds(v * vt, rows), sh * 2 + b, :],
                    sem,
                )
                for sh in range(n_sh)
            ]

        def step(buf, sem, tail):
            @pl.when(v >= 1)
            def _():
                for c in copies(buf, sem, vt):
                    c.wait()

            @pl.when(v < n_vt - 1)
            def _():
                for h in range(2):
                    buf[:, h * half:(h + 1) * half] = dot_half(h)
                for c in copies(buf, sem, vt):
                    c.start()

            @pl.when(v == n_vt - 1)
            def _():
                for h in range(2):
                    tail[:, h * half:(h + 1) * half] = dot_half(h)[:rem, :]
                for c in copies(tail, sem, rem):
                    c.start()

        @pl.when(b == 0)
        def _():
            step(res_a, sem_a, tail_a)

        @pl.when(b == 1)
        def _():
            step(res_b, sem_b, tail_b)

        @pl.when((v == n_vt - 1) & (b == 1))
        def _():
            for c in copies(tail_a, sem_a, rem):
                c.wait()
            for c in copies(tail_b, sem_b, rem):
                c.wait()

    out3 = pl.pallas_call(
        mm_kernel,
        grid=(n_vt, batch),
        in_specs=[
            pl.BlockSpec((n, emb), lambda v, b: (0, 0)),
            pl.BlockSpec((vt, emb),
                         lambda v, b: (jnp.minimum(v + b, n_vt - 1), 0)),
        ],
        out_specs=pl.BlockSpec(memory_space=pltpu.HBM),
        out_shape=jax.ShapeDtypeStruct((vocab, 2 * n_sh, 128),
                                       jnp.float32),
        scratch_shapes=[
            pltpu.VMEM((2 * vt, emb), jnp.bfloat16),
            pltpu.VMEM((vt, seq), jnp.float32),
            pltpu.VMEM((vt, seq), jnp.float32),
            pltpu.VMEM((rem, seq), jnp.float32),
            pltpu.VMEM((rem, seq), jnp.float32),
            pltpu.SemaphoreType.DMA,
            pltpu.SemaphoreType.DMA,
        ],
        compiler_params=pltpu.CompilerParams(
            dimension_semantics=("arbitrary", "arbitrary"),
        ),
    )(xs, w_out)
    out4 = out3.reshape(vocab, n_sh, batch, 128)
    return out4.transpose(2, 1, 3, 0).reshape(batch, seq, vocab)


def kernel(in_idx, tok_emb, pos_emb, W_out):
    batch, seq = in_idx.shape
    vocab, emb = W_out.shape
    idx_flat = in_idx.reshape(-1)
    x_tok = _sc_gather(tok_emb, idx_flat, batch * seq, emb)
    xs = _tc_prep(x_tok, pos_emb[:seq], batch, seq)
    return _tc_matmul(xs, W_out, batch, seq, vt=1024)
